# Initial kernel scaffold; baseline (speedup 1.0000x reference)
import functools
import jax
import jax.numpy as jnp
from jax import lax
from jax.experimental import pallas as pl
from jax.experimental.pallas import tpu as pltpu
from jax.experimental.pallas import tpu_sc as plsc

N = 10000
E = 320000
NC, NS, LL = 2, 16, 16
NW = NC * NS
EW = E // NW
KCH = 80
NCHUNK = EW // KCH
NP = 10240
NT = N // NS
GSEG = 256
ROWB = 400

_mesh = plsc.VectorSubcoreMesh(core_axis_name="c", subcore_axis_name="s",
                               num_cores=NC, num_subcores=NS)


@functools.partial(
    pl.kernel,
    out_type=jax.ShapeDtypeStruct((NC, NP), jnp.float32),
    mesh=_mesh,
    scratch_types=[
        pltpu.VMEM((EW,), jnp.int32),
        pltpu.VMEM((NP,), jnp.float32),
        pltpu.VMEM((NS, NP // NS), jnp.float32),
        pltpu.VMEM((NP // NS,), jnp.float32),
        pltpu.VMEM_SHARED((NS, NP), jnp.float32),
    ],
)
def _deg_kernel(row_hbm, out_hbm, ridx, cnt, comb, res, shared):
    c = lax.axis_index("c")
    s = lax.axis_index("s")
    wid = s * NC + c
    z16 = jnp.zeros((LL,), jnp.float32)

    def zb(i, _):
        cnt[pl.ds(i * LL, LL)] = z16
        return 0
    lax.fori_loop(0, NP // LL, zb, 0)

    pltpu.sync_copy(row_hbm.at[pl.ds(wid * EW, EW)], ridx)

    def body(i, _):
        r = ridx[i]
        cnt[r] = cnt[r] + 1.0
        return 0
    lax.fori_loop(0, EW, body, 0)

    pltpu.sync_copy(cnt, shared.at[s])
    plsc.subcore_barrier()
    w = NP // NS
    pltpu.sync_copy(shared.at[:, pl.ds(s * w, w)], comb)

    def rb(j, _):
        acc = comb[0, pl.ds(j * LL, LL)]
        for t in range(1, NS):
            acc = acc + comb[t, pl.ds(j * LL, LL)]
        res[pl.ds(j * LL, LL)] = acc
        return 0
    lax.fori_loop(0, w // LL, rb, 0)
    pltpu.sync_copy(res, out_hbm.at[c, pl.ds(s * w, w)])


@functools.partial(
    pl.kernel,
    out_type=jax.ShapeDtypeStruct((NC, N, 16), jnp.float32),
    mesh=_mesh,
    scratch_types=[
        pltpu.VMEM((N, 1), jnp.float32),
        pltpu.VMEM((KCH,), jnp.int32),
        pltpu.VMEM((KCH,), jnp.int32),
        pltpu.VMEM((KCH, 2), jnp.int32),
        pltpu.VMEM((KCH, 16), jnp.float32),
        pltpu.VMEM((NT, 16), jnp.float32),
        pltpu.VMEM_SHARED((N, 16), jnp.float32),
    ],
)
def _t_kernel(row_hbm, col_hbm, ea_hbm, dis_hbm, out_hbm,
              disv, ridx, cidx, eab, msg, zbuf, tsh):
    c = lax.axis_index("c")
    s = lax.axis_index("s")
    wid = s * NC + c
    lanes = jnp.arange(LL, dtype=jnp.int32)
    zi = jnp.zeros((LL,), jnp.int32)
    oi = jnp.full((LL,), 1, jnp.int32)
    z16 = jnp.zeros((LL,), jnp.float32)

    pltpu.sync_copy(dis_hbm, disv)

    def zrow(i, _):
        zbuf[i, :] = z16
        return 0
    lax.fori_loop(0, NT, zrow, 0)

    def zmsg(i, _):
        msg[i, :] = z16
        return 0
    lax.fori_loop(0, KCH, zmsg, 0)

    pltpu.sync_copy(zbuf, tsh.at[pl.ds(s * NT, NT)])
    plsc.subcore_barrier()

    def chunk(i, _):
        base = wid * EW + i * KCH
        pltpu.sync_copy(row_hbm.at[pl.ds(base, KCH)], ridx)
        pltpu.sync_copy(col_hbm.at[pl.ds(base, KCH)], cidx)
        pltpu.sync_copy(ea_hbm.at[pl.ds(base, KCH)], eab)
        saved = []
        for g in range(KCH // LL):
            lid = lanes + (g * LL)
            rv = ridx[pl.ds(g * LL, LL)]
            dv = plsc.load_gather(disv, [rv, zi])
            e0 = plsc.load_gather(eab, [lid, zi])
            e1 = plsc.load_gather(eab, [lid, oi])
            plsc.store_scatter(msg, [lid, e0], dv)
            plsc.store_scatter(msg, [lid, e1 + 6], dv)
            saved.append((lid, e0, e1))
        pltpu.sync_copy(msg, tsh.at[cidx], add=True)
        for (lid, e0, e1) in saved:
            plsc.store_scatter(msg, [lid, e0], z16)
            plsc.store_scatter(msg, [lid, e1 + 6], z16)
        return 0
    lax.fori_loop(0, NCHUNK, chunk, 0)

    plsc.subcore_barrier()
    pltpu.sync_copy(tsh.at[pl.ds(s * NT, NT)],
                    out_hbm.at[c, pl.ds(s * NT, NT)])


def _make_spmm(d):
    zr = 125

    @functools.partial(
        pl.kernel,
        out_type=jax.ShapeDtypeStruct((NC, N, d), jnp.float32),
        mesh=_mesh,
        scratch_types=[
            pltpu.VMEM((KCH,), jnp.int32),
            pltpu.VMEM((KCH,), jnp.int32),
            pltpu.VMEM((KCH, d), jnp.float32),
            pltpu.VMEM((zr, d), jnp.float32),
            pltpu.VMEM_SHARED((N, d), jnp.float32),
            pltpu.SemaphoreType.DMA,
        ],
    )
    def _spmm(hxp_hbm, row_hbm, col_hbm, out_hbm,
              ridx, cidx, rows, zbuf, acc, sem):
        c = lax.axis_index("c")
        s = lax.axis_index("s")
        wid = s * NC + c
        z16 = jnp.zeros((LL,), jnp.float32)

        def zrow(i, _):
            for j in range(d // LL):
                zbuf[i, pl.ds(j * LL, LL)] = z16
            return 0
        lax.fori_loop(0, zr, zrow, 0)
        for r in range(NT // zr):
            pltpu.sync_copy(zbuf, acc.at[pl.ds(s * NT + r * zr, zr)])
        plsc.subcore_barrier()

        def chunk(i, _):
            base = wid * EW + i * KCH
            pltpu.sync_copy(row_hbm.at[pl.ds(base, KCH)], ridx)
            pltpu.sync_copy(col_hbm.at[pl.ds(base, KCH)], cidx)
            pltpu.async_copy(hxp_hbm.at[ridx], rows, sem).wait()
            pltpu.sync_copy(rows, acc.at[cidx], add=True)
            return 0
        lax.fori_loop(0, NCHUNK, chunk, 0)

        plsc.subcore_barrier()
        pltpu.sync_copy(acc.at[pl.ds(s * NT, NT)],
                        out_hbm.at[c, pl.ds(s * NT, NT)])

    return _spmm


@functools.partial(
    pl.kernel,
    out_type=jax.ShapeDtypeStruct((N, 1), jnp.float32),
    mesh=_mesh,
    scratch_types=[
        pltpu.VMEM((640, 1), jnp.float32),
        pltpu.VMEM((640,), jnp.int32),
        pltpu.VMEM((640,), jnp.float32),
        pltpu.VMEM((640, 1), jnp.float32),
        pltpu.VMEM((GSEG,), jnp.float32),
        pltpu.VMEM((GSEG,), jnp.float32),
        pltpu.VMEM((GSEG,), jnp.float32),
        pltpu.VMEM((GSEG,), jnp.float32),
        pltpu.VMEM((NS, GSEG), jnp.float32),
        pltpu.VMEM_SHARED((NS, GSEG), jnp.float32),
        pltpu.VMEM_SHARED((NS, GSEG), jnp.float32),
    ],
)
def _softmax_kernel(nr_hbm, batch_hbm, out_hbm,
                    vbuf, bbuf, exbuf, obuf, smax, ssum, gmax, gsum,
                    call, shmax, shsum):
    c = lax.axis_index("c")
    s = lax.axis_index("s")
    lanes = jnp.arange(LL, dtype=jnp.int32)
    zi = jnp.zeros((LL,), jnp.int32)
    z16 = jnp.zeros((LL,), jnp.float32)
    ninf = jnp.full((LL,), -3.0e38, jnp.float32)

    for g in range(640 // LL):
        bbuf[pl.ds(g * LL, LL)] = zi
        plsc.store_scatter(vbuf, [lanes + (g * LL), zi], ninf)
    pltpu.sync_copy(nr_hbm.at[pl.ds(s * NT, NT)], vbuf.at[pl.ds(0, NT)])
    pltpu.sync_copy(batch_hbm.at[pl.ds(s * NT, NT)], bbuf.at[pl.ds(0, NT)])

    for g in range(GSEG // LL):
        smax[pl.ds(g * LL, LL)] = ninf

    def mloop(i, _):
        b = bbuf[i]
        v = vbuf[i, 0]
        smax[b] = jnp.maximum(smax[b], v)
        return 0
    lax.fori_loop(0, NT, mloop, 0)

    pltpu.sync_copy(smax, shmax.at[s])
    plsc.subcore_barrier()
    pltpu.sync_copy(shmax, call)
    for g in range(GSEG // LL):
        acc = call[0, pl.ds(g * LL, LL)]
        for t in range(1, NS):
            acc = jnp.maximum(acc, call[t, pl.ds(g * LL, LL)])
        gmax[pl.ds(g * LL, LL)] = acc

    for g in range(640 // LL):
        lid = lanes + (g * LL)
        vv = plsc.load_gather(vbuf, [lid, zi])
        bv = bbuf[pl.ds(g * LL, LL)]
        mg = plsc.load_gather(gmax, [bv])
        exbuf[pl.ds(g * LL, LL)] = jnp.exp(vv - mg)

    for g in range(GSEG // LL):
        ssum[pl.ds(g * LL, LL)] = z16

    def sloop(i, _):
        b = bbuf[i]
        ssum[b] = ssum[b] + exbuf[i]
        return 0
    lax.fori_loop(0, NT, sloop, 0)

    pltpu.sync_copy(ssum, shsum.at[s])
    plsc.subcore_barrier()
    pltpu.sync_copy(shsum, call)
    for g in range(GSEG // LL):
        acc = call[0, pl.ds(g * LL, LL)]
        for t in range(1, NS):
            acc = acc + call[t, pl.ds(g * LL, LL)]
        gsum[pl.ds(g * LL, LL)] = acc

    for g in range(640 // LL):
        lid = lanes + (g * LL)
        bv = bbuf[pl.ds(g * LL, LL)]
        sv = plsc.load_gather(gsum, [bv])
        ev = exbuf[pl.ds(g * LL, LL)]
        plsc.store_scatter(obuf, [lid, zi], ev / (sv + 1e-16))

    @pl.when(jnp.logical_and(s >= c * 8, s < c * 8 + 8))
    def _():
        pltpu.sync_copy(obuf.at[pl.ds(0, NT)], out_hbm.at[pl.ds(s * NT, NT)])


def _embed_call(x, cnt2, xe1, xe2, w1, b1):
    v1 = xe1.shape[0]
    v2 = xe2.shape[0]
    emb = xe1.shape[1]
    d1 = w1.shape[1]

    def body(x_ref, cnt_ref, xe1_ref, xe2_ref, w_ref, b_ref, hxp_ref, dis_ref):
        xb = x_ref[...]
        deg = cnt_ref[0, :] + cnt_ref[1, :] + 1.0
        dis = lax.rsqrt(deg)[:, None]
        oh0 = (xb[:, 0:1] == lax.broadcasted_iota(jnp.int32, (1, v1), 1)
               ).astype(jnp.float32)
        oh1 = (xb[:, 1:2] == lax.broadcasted_iota(jnp.int32, (1, v2), 1)
               ).astype(jnp.float32)
        h0 = (jnp.dot(oh0, xe1_ref[...], preferred_element_type=jnp.float32)
              + jnp.dot(oh1, xe2_ref[...], preferred_element_type=jnp.float32))
        hx = jnp.dot(h0, w_ref[...], preferred_element_type=jnp.float32) \
            + b_ref[...]
        hxp_ref[...] = dis * hx
        dis_ref[...] = dis

    return pl.pallas_call(
        body,
        grid=(N // ROWB,),
        in_specs=[
            pl.BlockSpec((ROWB, 2), lambda i: (i, 0)),
            pl.BlockSpec((NC, ROWB), lambda i: (0, i)),
            pl.BlockSpec((v1, emb), lambda i: (0, 0)),
            pl.BlockSpec((v2, emb), lambda i: (0, 0)),
            pl.BlockSpec((emb, d1), lambda i: (0, 0)),
            pl.BlockSpec((1, d1), lambda i: (0, 0)),
        ],
        out_specs=[
            pl.BlockSpec((ROWB, d1), lambda i: (i, 0)),
            pl.BlockSpec((ROWB, 1), lambda i: (i, 0)),
        ],
        out_shape=[
            jax.ShapeDtypeStruct((N, d1), jnp.float32),
            jax.ShapeDtypeStruct((N, 1), jnp.float32),
        ],
    )(x, cnt2, xe1, xe2, w1, b1)


def _zstats_call(acc2, hxp, dis, t2, ecat):
    d = hxp.shape[1]

    def body(acc_ref, hxp_ref, dis_ref, t_ref, ec_ref, z_ref, st_ref):
        i = pl.program_id(0)
        dis_b = dis_ref[...]
        ec = ec_ref[...]
        tb = t_ref[0] + t_ref[1]
        ee = jnp.dot(tb, ec, preferred_element_type=jnp.float32)
        sl = (ec[4, :] + ec[6, :])[None, :]
        z = dis_b * (acc_ref[0] + acc_ref[1] + hxp_ref[...] + ee + dis_b * sl)
        z_ref[...] = z
        ps = jnp.concatenate(
            [jnp.sum(z, axis=0, keepdims=True),
             jnp.sum(z * z, axis=0, keepdims=True)], axis=0)
        st_ref[...] = jnp.where(i == 0, ps, st_ref[...] + ps)

    return pl.pallas_call(
        body,
        grid=(N // ROWB,),
        in_specs=[
            pl.BlockSpec((NC, ROWB, d), lambda i: (0, i, 0)),
            pl.BlockSpec((ROWB, d), lambda i: (i, 0)),
            pl.BlockSpec((ROWB, 1), lambda i: (i, 0)),
            pl.BlockSpec((NC, ROWB, 16), lambda i: (0, i, 0)),
            pl.BlockSpec((16, d), lambda i: (0, 0)),
        ],
        out_specs=[
            pl.BlockSpec((ROWB, d), lambda i: (i, 0)),
            pl.BlockSpec((2, d), lambda i: (0, 0)),
        ],
        out_shape=[
            jax.ShapeDtypeStruct((N, d), jnp.float32),
            jax.ShapeDtypeStruct((2, d), jnp.float32),
        ],
    )(acc2, hxp, dis, t2, ecat)


def _bn_mm_call(z, stats, gamma, beta, wn, bn, dis, relu, final):
    d = z.shape[1]
    dn = wn.shape[1]
    inv_n = 1.0 / float(N)

    def body(z_ref, st_ref, g_ref, be_ref, w_ref, b_ref, dis_ref, out_ref):
        st = st_ref[...]
        mean = st[0:1, :] * inv_n
        var = st[1:2, :] * inv_n - mean * mean
        scale = g_ref[...] * lax.rsqrt(var + 1e-5)
        shift = be_ref[...] - mean * scale
        h = z_ref[...] * scale + shift
        if relu:
            h = jnp.maximum(h, 0.0)
        o = jnp.dot(h, w_ref[...], preferred_element_type=jnp.float32) \
            + b_ref[...]
        out_ref[...] = o if final else dis_ref[...] * o

    return pl.pallas_call(
        body,
        grid=(N // ROWB,),
        in_specs=[
            pl.BlockSpec((ROWB, d), lambda i: (i, 0)),
            pl.BlockSpec((2, d), lambda i: (0, 0)),
            pl.BlockSpec((1, d), lambda i: (0, 0)),
            pl.BlockSpec((1, d), lambda i: (0, 0)),
            pl.BlockSpec((d, dn), lambda i: (0, 0)),
            pl.BlockSpec((1, dn), lambda i: (0, 0)),
            pl.BlockSpec((ROWB, 1), lambda i: (i, 0)),
        ],
        out_specs=pl.BlockSpec((ROWB, dn), lambda i: (i, 0)),
        out_shape=jax.ShapeDtypeStruct((N, dn), jnp.float32),
    )(z, stats, gamma, beta, wn, bn, dis)


_spmm_cache = {}


def _spmm_call(hxp, row, col):
    d = hxp.shape[1]
    if d not in _spmm_cache:
        _spmm_cache[d] = _make_spmm(d)
    return _spmm_cache[d](hxp, row, col)


def kernel(x, edge_index, edge_attr, batch, params):
    row = edge_index[0]
    col = edge_index[1]
    layers = params['layers']

    cnt2 = _deg_kernel(row)

    l0 = layers[0]
    hxp, dis = _embed_call(x, cnt2, params['xe1'], params['xe2'],
                           l0['W'], l0['b'].reshape(1, -1))

    t2 = _t_kernel(row, col, edge_attr, dis)

    nl = len(layers)
    nr = None
    for li in range(nl):
        p = layers[li]
        d = p['W'].shape[1]
        acc2 = _spmm_call(hxp, row, col)
        ecat = jnp.concatenate(
            [p['e1'], p['e2'],
             jnp.zeros((16 - p['e1'].shape[0] - p['e2'].shape[0], d),
                       jnp.float32)], axis=0)
        z, stats = _zstats_call(acc2, hxp, dis, t2, ecat)
        if li < nl - 1:
            pn = layers[li + 1]
            hxp = _bn_mm_call(z, stats, p['gamma'].reshape(1, -1),
                              p['beta'].reshape(1, -1), pn['W'],
                              pn['b'].reshape(1, -1), dis,
                              relu=True, final=False)
        else:
            nr = _bn_mm_call(z, stats, p['gamma'].reshape(1, -1),
                             p['beta'].reshape(1, -1), params['Wf'],
                             params['bf'].reshape(1, -1), dis,
                             relu=False, final=True)

    return _softmax_kernel(nr, batch)


# speedup vs baseline: 8.5867x; 8.5867x over previous
import functools
import jax
import jax.numpy as jnp
from jax import lax
from jax.experimental import pallas as pl
from jax.experimental.pallas import tpu as pltpu
from jax.experimental.pallas import tpu_sc as plsc

N = 10000
E = 320000
NC, NS, LL = 2, 16, 16
NW = NC * NS
EW = E // NW
KCH = 80
NCHUNK = EW // KCH
NP = 10240
BS = NP // NS
GSEG = 256
ROWB = 400

_mesh = plsc.VectorSubcoreMesh(core_axis_name="c", subcore_axis_name="s",
                               num_cores=NC, num_subcores=NS)


@functools.partial(
    pl.kernel,
    out_type=jax.ShapeDtypeStruct((NC * NP,), jnp.float32),
    mesh=_mesh,
    compiler_params=pltpu.CompilerParams(use_tc_tiling_on_sc=False, needs_layout_passes=False),
    scratch_types=[
        pltpu.VMEM((EW + LL,), jnp.int32),
        pltpu.VMEM((NP + LL,), jnp.float32),
        pltpu.VMEM((NS, BS), jnp.float32),
        pltpu.VMEM((BS,), jnp.float32),
        pltpu.VMEM_SHARED((NS, NP), jnp.float32),
    ],
)
def _deg_kernel(row_hbm, out_hbm, ridx, cnt, comb, res, shared):
    c = lax.axis_index("c")
    s = lax.axis_index("s")
    wid = s * NC + c
    z16 = jnp.zeros((LL,), jnp.float32)
    lanes = jnp.arange(LL, dtype=jnp.int32)
    oneh = jnp.where(lanes == 0, 1.0, 0.0).astype(jnp.float32)

    def zb(i, _):
        cnt[pl.ds(i * LL, LL)] = z16
        return 0
    lax.fori_loop(0, (NP + LL) // LL, zb, 0)

    pltpu.sync_copy(row_hbm.at[pl.ds(wid * EW, EW)], ridx.at[pl.ds(0, EW)])

    def body(i, _):
        r = ridx[pl.ds(i, LL)][0]
        w = cnt[pl.ds(r, LL)]
        cnt[pl.ds(r, LL)] = w + oneh
        return 0
    lax.fori_loop(0, EW, body, 0)

    pltpu.sync_copy(cnt.at[pl.ds(0, NP)], shared.at[s])
    plsc.subcore_barrier()
    pltpu.sync_copy(shared.at[:, pl.ds(s * BS, BS)], comb)

    for j in range(BS // LL):
        acc = comb[0, pl.ds(j * LL, LL)]
        for t in range(1, NS):
            acc = acc + comb[t, pl.ds(j * LL, LL)]
        res[pl.ds(j * LL, LL)] = acc
    pltpu.sync_copy(res, out_hbm.at[pl.ds(c * NP + s * BS, BS)])


@functools.partial(
    pl.kernel,
    out_type=jax.ShapeDtypeStruct((NC, NP, 16), jnp.float32),
    mesh=_mesh,
    compiler_params=pltpu.CompilerParams(use_tc_tiling_on_sc=False, needs_layout_passes=False),
    scratch_types=[
        pltpu.VMEM((NP,), jnp.float32),
        pltpu.VMEM((KCH,), jnp.int32),
        pltpu.VMEM((KCH,), jnp.int32),
        pltpu.VMEM((KCH, 2), jnp.int32),
        pltpu.VMEM((KCH, 16), jnp.float32),
        pltpu.VMEM((BS, 16), jnp.float32),
        pltpu.VMEM_SHARED((NP, 16), jnp.float32),
    ],
)
def _t_kernel(row_hbm, col_hbm, ea_hbm, dis_hbm, out_hbm,
              disv, ridx, cidx, eab, msg, zbuf, tsh):
    c = lax.axis_index("c")
    s = lax.axis_index("s")
    wid = s * NC + c
    lanes = jnp.arange(LL, dtype=jnp.int32)
    zi = jnp.zeros((LL,), jnp.int32)
    oi = jnp.full((LL,), 1, jnp.int32)
    z16 = jnp.zeros((LL,), jnp.float32)

    pltpu.sync_copy(dis_hbm, disv.at[pl.ds(0, N)])

    def zrow(i, _):
        zbuf[i, :] = z16
        return 0
    lax.fori_loop(0, BS, zrow, 0)

    def zmsg(i, _):
        msg[i, :] = z16
        return 0
    lax.fori_loop(0, KCH, zmsg, 0)

    pltpu.sync_copy(zbuf, tsh.at[pl.ds(s * BS, BS)])
    plsc.subcore_barrier()

    def chunk(i, _):
        base = wid * EW + i * KCH
        pltpu.sync_copy(row_hbm.at[pl.ds(base, KCH)], ridx)
        pltpu.sync_copy(col_hbm.at[pl.ds(base, KCH)], cidx)
        pltpu.sync_copy(ea_hbm.at[pl.ds(base, KCH)], eab)
        saved = []
        for g in range(KCH // LL):
            lid = lanes + (g * LL)
            rv = ridx[pl.ds(g * LL, LL)]
            dv = plsc.load_gather(disv, [rv])
            e0 = plsc.load_gather(eab, [lid, zi])
            e1 = plsc.load_gather(eab, [lid, oi])
            plsc.store_scatter(msg, [lid, e0], dv)
            plsc.store_scatter(msg, [lid, e1 + 6], dv)
            saved.append((lid, e0, e1))
        pltpu.sync_copy(msg, tsh.at[cidx], add=True)
        for (lid, e0, e1) in saved:
            plsc.store_scatter(msg, [lid, e0], z16)
            plsc.store_scatter(msg, [lid, e1 + 6], z16)
        return 0
    lax.fori_loop(0, NCHUNK, chunk, 0)

    plsc.subcore_barrier()
    pltpu.sync_copy(tsh.at[pl.ds(s * BS, BS)],
                    out_hbm.at[c, pl.ds(s * BS, BS)])


def _make_spmm(d):
    zr = 128

    @functools.partial(
        pl.kernel,
        out_type=jax.ShapeDtypeStruct((NC, NP, d), jnp.float32),
        mesh=_mesh,
        compiler_params=pltpu.CompilerParams(use_tc_tiling_on_sc=False, needs_layout_passes=False),
        scratch_types=[
            pltpu.VMEM((KCH,), jnp.int32),
            pltpu.VMEM((KCH,), jnp.int32),
            pltpu.VMEM((KCH, d), jnp.float32),
            pltpu.VMEM((zr, d), jnp.float32),
            pltpu.VMEM_SHARED((NP, d), jnp.float32),
            pltpu.SemaphoreType.DMA,
        ],
    )
    def _spmm(hxp_hbm, row_hbm, col_hbm, out_hbm,
              ridx, cidx, rows, zbuf, acc, sem):
        c = lax.axis_index("c")
        s = lax.axis_index("s")
        wid = s * NC + c
        z16 = jnp.zeros((LL,), jnp.float32)

        def zrow(i, _):
            for j in range(d // LL):
                zbuf[i, pl.ds(j * LL, LL)] = z16
            return 0
        lax.fori_loop(0, zr, zrow, 0)
        for r in range(BS // zr):
            pltpu.sync_copy(zbuf, acc.at[pl.ds(s * BS + r * zr, zr)])
        plsc.subcore_barrier()

        def chunk(i, _):
            base = wid * EW + i * KCH
            pltpu.sync_copy(row_hbm.at[pl.ds(base, KCH)], ridx)
            pltpu.sync_copy(col_hbm.at[pl.ds(base, KCH)], cidx)
            pltpu.async_copy(hxp_hbm.at[ridx], rows, sem).wait()
            pltpu.sync_copy(rows, acc.at[cidx], add=True)
            return 0
        lax.fori_loop(0, NCHUNK, chunk, 0)

        plsc.subcore_barrier()
        pltpu.sync_copy(acc.at[pl.ds(s * BS, BS)],
                        out_hbm.at[c, pl.ds(s * BS, BS)])

    return _spmm


@functools.partial(
    pl.kernel,
    out_type=jax.ShapeDtypeStruct((NP,), jnp.float32),
    mesh=_mesh,
    compiler_params=pltpu.CompilerParams(use_tc_tiling_on_sc=False, needs_layout_passes=False),
    scratch_types=[
        pltpu.VMEM((BS + LL,), jnp.int32),
        pltpu.VMEM((BS + LL,), jnp.float32),
        pltpu.VMEM((BS + LL,), jnp.float32),
        pltpu.VMEM((BS,), jnp.float32),
        pltpu.VMEM((GSEG + 128,), jnp.float32),
        pltpu.VMEM((GSEG + 128,), jnp.float32),
        pltpu.VMEM((GSEG,), jnp.float32),
        pltpu.VMEM((GSEG,), jnp.float32),
        pltpu.VMEM((NS, GSEG), jnp.float32),
        pltpu.VMEM_SHARED((NS, GSEG), jnp.float32),
        pltpu.VMEM_SHARED((NS, GSEG), jnp.float32),
    ],
)
def _softmax_kernel(nr_hbm, batch_hbm, out_hbm,
                    bbuf, vflat, exbuf, obuf, smax, ssum, gmax, gsum,
                    call, shmax, shsum):
    c = lax.axis_index("c")
    s = lax.axis_index("s")
    lanes = jnp.arange(LL, dtype=jnp.int32)
    z16 = jnp.zeros((LL,), jnp.float32)
    ninf = jnp.full((LL,), -3.0e38, jnp.float32)
    l0m = lanes == 0

    pltpu.sync_copy(nr_hbm.at[pl.ds(s * BS, BS)], vflat.at[pl.ds(0, BS)])
    pltpu.sync_copy(batch_hbm.at[pl.ds(s * BS, BS)], bbuf.at[pl.ds(0, BS)])

    for g in range((GSEG + LL) // LL):
        smax[pl.ds(g * LL, LL)] = ninf

    def mloop(i, _):
        b = bbuf[pl.ds(i, LL)][0]
        v = vflat[pl.ds(i, LL)][0]
        w = smax[pl.ds(b, LL)]
        smax[pl.ds(b, LL)] = jnp.where(
            l0m, jnp.maximum(w, lax.full((LL,), v, jnp.float32)), w)
        return 0
    lax.fori_loop(0, BS, mloop, 0)

    pltpu.sync_copy(smax.at[pl.ds(0, GSEG)], shmax.at[s])
    plsc.subcore_barrier()
    pltpu.sync_copy(shmax, call)
    for g in range(GSEG // LL):
        acc = call[0, pl.ds(g * LL, LL)]
        for t in range(1, NS):
            acc = jnp.maximum(acc, call[t, pl.ds(g * LL, LL)])
        gmax[pl.ds(g * LL, LL)] = acc

    for g in range(BS // LL):
        vv = vflat[pl.ds(g * LL, LL)]
        bv = bbuf[pl.ds(g * LL, LL)]
        mg = plsc.load_gather(gmax, [bv])
        exbuf[pl.ds(g * LL, LL)] = jnp.exp(vv - mg)

    for g in range((GSEG + LL) // LL):
        ssum[pl.ds(g * LL, LL)] = z16

    def sloop(i, _):
        b = bbuf[pl.ds(i, LL)][0]
        ev = exbuf[pl.ds(i, LL)][0]
        w = ssum[pl.ds(b, LL)]
        ssum[pl.ds(b, LL)] = jnp.where(
            l0m, w + lax.full((LL,), ev, jnp.float32), w)
        return 0
    lax.fori_loop(0, BS, sloop, 0)

    pltpu.sync_copy(ssum.at[pl.ds(0, GSEG)], shsum.at[s])
    plsc.subcore_barrier()
    pltpu.sync_copy(shsum, call)
    for g in range(GSEG // LL):
        acc = call[0, pl.ds(g * LL, LL)]
        for t in range(1, NS):
            acc = acc + call[t, pl.ds(g * LL, LL)]
        gsum[pl.ds(g * LL, LL)] = acc

    for g in range(BS // LL):
        bv = bbuf[pl.ds(g * LL, LL)]
        sv = plsc.load_gather(gsum, [bv])
        ev = exbuf[pl.ds(g * LL, LL)]
        obuf[pl.ds(g * LL, LL)] = ev / (sv + 1e-16)

    @pl.when(jnp.logical_and(s >= c * 8, s < c * 8 + 8))
    def _():
        pltpu.sync_copy(obuf, out_hbm.at[pl.ds(s * BS, BS)])


def _embed_call(x, cnt2, xe1, xe2, w1, b1):
    v1 = xe1.shape[0]
    v2 = xe2.shape[0]
    emb = xe1.shape[1]
    d1 = w1.shape[1]

    def body(x_ref, cnt_ref, xe1_ref, xe2_ref, w_ref, b_ref, hxp_ref, dis_ref):
        i = pl.program_id(0)
        xb = x_ref[...]
        cb = cnt_ref[:, pl.ds(i * ROWB, ROWB), :]
        deg = cb[0] + cb[1] + 1.0
        dis = lax.rsqrt(deg)
        oh0 = (xb[:, 0:1] == lax.broadcasted_iota(jnp.int32, (1, v1), 1)
               ).astype(jnp.float32)
        oh1 = (xb[:, 1:2] == lax.broadcasted_iota(jnp.int32, (1, v2), 1)
               ).astype(jnp.float32)
        h0 = (jnp.dot(oh0, xe1_ref[...], preferred_element_type=jnp.float32)
              + jnp.dot(oh1, xe2_ref[...], preferred_element_type=jnp.float32))
        hx = jnp.dot(h0, w_ref[...], preferred_element_type=jnp.float32) \
            + b_ref[...]
        hxp_ref[...] = dis * hx
        dis_ref[...] = dis

    return pl.pallas_call(
        body,
        grid=(N // ROWB,),
        in_specs=[
            pl.BlockSpec((ROWB, 2), lambda i: (i, 0)),
            pl.BlockSpec((NC, NP, 1), lambda i: (0, 0, 0)),
            pl.BlockSpec((v1, emb), lambda i: (0, 0)),
            pl.BlockSpec((v2, emb), lambda i: (0, 0)),
            pl.BlockSpec((emb, d1), lambda i: (0, 0)),
            pl.BlockSpec((1, d1), lambda i: (0, 0)),
        ],
        out_specs=[
            pl.BlockSpec((ROWB, d1), lambda i: (i, 0)),
            pl.BlockSpec((ROWB, 1), lambda i: (i, 0)),
        ],
        out_shape=[
            jax.ShapeDtypeStruct((N, d1), jnp.float32),
            jax.ShapeDtypeStruct((N, 1), jnp.float32),
        ],
    )(x, cnt2, xe1, xe2, w1, b1)


def _zstats_call(acc2, hxp, dis, t2, ecat):
    d = hxp.shape[1]

    def body(acc_ref, hxp_ref, dis_ref, t_ref, ec_ref, z_ref, st_ref):
        i = pl.program_id(0)
        dis_b = dis_ref[...]
        ec = ec_ref[...]
        tb = t_ref[0] + t_ref[1]
        ee = jnp.dot(tb, ec, preferred_element_type=jnp.float32)
        sl = (ec[4, :] + ec[6, :])[None, :]
        z = dis_b * (acc_ref[0] + acc_ref[1] + hxp_ref[...] + ee + dis_b * sl)
        z_ref[...] = z
        ps = jnp.concatenate(
            [jnp.sum(z, axis=0, keepdims=True),
             jnp.sum(z * z, axis=0, keepdims=True)], axis=0)
        st_ref[...] = jnp.where(i == 0, ps, st_ref[...] + ps)

    return pl.pallas_call(
        body,
        grid=(N // ROWB,),
        in_specs=[
            pl.BlockSpec((NC, ROWB, d), lambda i: (0, i, 0)),
            pl.BlockSpec((ROWB, d), lambda i: (i, 0)),
            pl.BlockSpec((ROWB, 1), lambda i: (i, 0)),
            pl.BlockSpec((NC, ROWB, 16), lambda i: (0, i, 0)),
            pl.BlockSpec((16, d), lambda i: (0, 0)),
        ],
        out_specs=[
            pl.BlockSpec((ROWB, d), lambda i: (i, 0)),
            pl.BlockSpec((2, d), lambda i: (0, 0)),
        ],
        out_shape=[
            jax.ShapeDtypeStruct((N, d), jnp.float32),
            jax.ShapeDtypeStruct((2, d), jnp.float32),
        ],
    )(acc2, hxp, dis, t2, ecat)


def _bn_mm_call(z, stats, gamma, beta, wn, bn, dis, relu, final):
    d = z.shape[1]
    dn = wn.shape[1]
    inv_n = 1.0 / float(N)

    def body(z_ref, st_ref, g_ref, be_ref, w_ref, b_ref, dis_ref, out_ref):
        st = st_ref[...]
        mean = st[0:1, :] * inv_n
        var = st[1:2, :] * inv_n - mean * mean
        scale = g_ref[...] * lax.rsqrt(var + 1e-5)
        shift = be_ref[...] - mean * scale
        h = z_ref[...] * scale + shift
        if relu:
            h = jnp.maximum(h, 0.0)
        o = jnp.dot(h, w_ref[...], preferred_element_type=jnp.float32) \
            + b_ref[...]
        out_ref[...] = o if final else dis_ref[...] * o

    return pl.pallas_call(
        body,
        grid=(N // ROWB,),
        in_specs=[
            pl.BlockSpec((ROWB, d), lambda i: (i, 0)),
            pl.BlockSpec((2, d), lambda i: (0, 0)),
            pl.BlockSpec((1, d), lambda i: (0, 0)),
            pl.BlockSpec((1, d), lambda i: (0, 0)),
            pl.BlockSpec((d, dn), lambda i: (0, 0)),
            pl.BlockSpec((1, dn), lambda i: (0, 0)),
            pl.BlockSpec((ROWB, 1), lambda i: (i, 0)),
        ],
        out_specs=pl.BlockSpec((ROWB, dn), lambda i: (i, 0)),
        out_shape=jax.ShapeDtypeStruct((N, dn), jnp.float32),
    )(z, stats, gamma, beta, wn, bn, dis)


_spmm_cache = {}


def _spmm_call(hxp, row, col):
    d = hxp.shape[1]
    if d not in _spmm_cache:
        _spmm_cache[d] = _make_spmm(d)
    return _spmm_cache[d](hxp, row, col)


def kernel(x, edge_index, edge_attr, batch, params):
    row = edge_index[0]
    col = edge_index[1]
    layers = params['layers']

    cnt2 = _deg_kernel(row).reshape(NC, NP, 1)

    l0 = layers[0]
    hxp, dis = _embed_call(x, cnt2, params['xe1'], params['xe2'],
                           l0['W'], l0['b'].reshape(1, -1))

    t2 = _t_kernel(row, col, edge_attr, dis[:, 0])

    nl = len(layers)
    nr = None
    for li in range(nl):
        p = layers[li]
        d = p['W'].shape[1]
        acc2 = _spmm_call(hxp, row, col)
        ecat = jnp.concatenate(
            [p['e1'], p['e2'],
             jnp.zeros((16 - p['e1'].shape[0] - p['e2'].shape[0], d),
                       jnp.float32)], axis=0)
        z, stats = _zstats_call(acc2, hxp, dis, t2, ecat)
        if li < nl - 1:
            pn = layers[li + 1]
            hxp = _bn_mm_call(z, stats, p['gamma'].reshape(1, -1),
                              p['beta'].reshape(1, -1), pn['W'],
                              pn['b'].reshape(1, -1), dis,
                              relu=True, final=False)
        else:
            nr = _bn_mm_call(z, stats, p['gamma'].reshape(1, -1),
                             p['beta'].reshape(1, -1), params['Wf'],
                             params['bf'].reshape(1, -1), dis,
                             relu=False, final=True)

    nrp = jnp.pad(nr[:, 0], (0, NP - N), constant_values=-3.0e38)
    batchp = jnp.pad(batch, (0, NP - N))
    return _softmax_kernel(nrp, batchp)[:N, None]


# speedup vs baseline: 9.2121x; 1.0728x over previous
import functools
import jax
import jax.numpy as jnp
from jax import lax
from jax.experimental import pallas as pl
from jax.experimental.pallas import tpu as pltpu
from jax.experimental.pallas import tpu_sc as plsc

N = 10000
E = 320000
NC, NS, LL = 2, 16, 16
NW = NC * NS
EW = E // NW
KCH = 80
NCHUNK = EW // KCH
NP = 10240
BS = NP // NS
GSEG = 256
ROWB = 400

_mesh = plsc.VectorSubcoreMesh(core_axis_name="c", subcore_axis_name="s",
                               num_cores=NC, num_subcores=NS)


@functools.partial(
    pl.kernel,
    out_type=jax.ShapeDtypeStruct((NC * NP,), jnp.float32),
    mesh=_mesh,
    compiler_params=pltpu.CompilerParams(use_tc_tiling_on_sc=False, needs_layout_passes=False),
    scratch_types=[
        pltpu.VMEM((EW + LL,), jnp.int32),
        pltpu.VMEM((NP + LL,), jnp.float32),
        pltpu.VMEM((NS, BS), jnp.float32),
        pltpu.VMEM((BS,), jnp.float32),
        pltpu.VMEM_SHARED((NS, NP), jnp.float32),
    ],
)
def _deg_kernel(row_hbm, out_hbm, ridx, cnt, comb, res, shared):
    c = lax.axis_index("c")
    s = lax.axis_index("s")
    wid = s * NC + c
    z16 = jnp.zeros((LL,), jnp.float32)
    lanes = jnp.arange(LL, dtype=jnp.int32)
    oneh = jnp.where(lanes == 0, 1.0, 0.0).astype(jnp.float32)

    def zb(i, _):
        cnt[pl.ds(i * LL, LL)] = z16
        return 0
    lax.fori_loop(0, (NP + LL) // LL, zb, 0)

    pltpu.sync_copy(row_hbm.at[pl.ds(wid * EW, EW)], ridx.at[pl.ds(0, EW)])

    def body(j, _):
        rv = ridx[pl.ds(j * LL, LL)]
        for l in range(LL):
            r = rv[l]
            w = cnt[pl.ds(r, LL)]
            cnt[pl.ds(r, LL)] = w + oneh
        return 0
    lax.fori_loop(0, EW // LL, body, 0)

    pltpu.sync_copy(cnt.at[pl.ds(0, NP)], shared.at[s])
    plsc.subcore_barrier()
    pltpu.sync_copy(shared.at[:, pl.ds(s * BS, BS)], comb)

    for j in range(BS // LL):
        acc = comb[0, pl.ds(j * LL, LL)]
        for t in range(1, NS):
            acc = acc + comb[t, pl.ds(j * LL, LL)]
        res[pl.ds(j * LL, LL)] = acc
    pltpu.sync_copy(res, out_hbm.at[pl.ds(c * NP + s * BS, BS)])


def _make_spmm(d, with_t):
    zr = 32
    out_types = [jax.ShapeDtypeStruct((NC, NP, d), jnp.float32)]
    scratch = [
        [pltpu.VMEM((KCH,), jnp.int32) for _ in range(4)],
        [pltpu.VMEM((KCH,), jnp.int32) for _ in range(4)],
        [pltpu.SemaphoreType.DMA for _ in range(4)],
        [pltpu.VMEM((KCH, d), jnp.float32) for _ in range(2)],
        [pltpu.SemaphoreType.DMA for _ in range(2)],
        pltpu.VMEM((zr, d), jnp.float32),
        pltpu.VMEM_SHARED((NP, d), jnp.float32),
    ]
    if with_t:
        out_types.append(jax.ShapeDtypeStruct((NC, NP, 16), jnp.float32))
        scratch += [
            pltpu.VMEM((NP,), jnp.float32),
            [pltpu.VMEM((KCH, 2), jnp.int32) for _ in range(4)],
            pltpu.VMEM((KCH, 16), jnp.float32),
            pltpu.VMEM_SHARED((NP, 16), jnp.float32),
        ]

    @functools.partial(
        pl.kernel,
        out_type=tuple(out_types),
        mesh=_mesh,
        compiler_params=pltpu.CompilerParams(use_tc_tiling_on_sc=False,
                                             needs_layout_passes=False),
        scratch_types=scratch,
    )
    def _spmm(hxp_hbm, row_hbm, col_hbm, *rest):
        if with_t:
            (ea_hbm, dis_hbm, out_hbm, tout_hbm, rbuf, cbuf, semi, rows,
             semg, zbuf, acc, disv, ebuf, msg, tsh) = rest
        else:
            (out_hbm, rbuf, cbuf, semi, rows, semg, zbuf, acc) = rest
        c = lax.axis_index("c")
        s = lax.axis_index("s")
        wid = s * NC + c
        lanes = jnp.arange(LL, dtype=jnp.int32)
        zi = jnp.zeros((LL,), jnp.int32)
        oi = jnp.full((LL,), 1, jnp.int32)
        z16 = jnp.zeros((LL,), jnp.float32)

        def idx_fill(cch, p):
            base = jnp.minimum(wid * EW + cch * KCH, E - KCH)
            pltpu.async_copy(row_hbm.at[pl.ds(base, KCH)], rbuf[p], semi[p])
            pltpu.async_copy(col_hbm.at[pl.ds(base, KCH)], cbuf[p], semi[p])
            if with_t:
                pltpu.async_copy(ea_hbm.at[pl.ds(base, KCH)],
                                 ebuf[p], semi[p])

        def idx_wait(p):
            pltpu.make_async_copy(row_hbm.at[pl.ds(0, KCH)],
                                  rbuf[p], semi[p]).wait()
            pltpu.make_async_copy(col_hbm.at[pl.ds(0, KCH)],
                                  cbuf[p], semi[p]).wait()
            if with_t:
                pltpu.make_async_copy(ea_hbm.at[pl.ds(0, KCH)],
                                      ebuf[p], semi[p]).wait()

        def gather(p, b):
            return pltpu.async_copy(hxp_hbm.at[rbuf[p]], rows[b], semg[b])

        if with_t:
            pltpu.sync_copy(dis_hbm, disv.at[pl.ds(0, N)])

        def zrow(i, _):
            for j in range(d // LL):
                zbuf[i, pl.ds(j * LL, LL)] = z16
            return 0
        lax.fori_loop(0, zr, zrow, 0)
        for r in range(BS // zr):
            pltpu.sync_copy(zbuf, acc.at[pl.ds(s * BS + r * zr, zr)])
        if with_t:
            def zmsg(i, _):
                msg[i, :] = z16
                return 0
            lax.fori_loop(0, KCH, zmsg, 0)
            for r in range(BS // KCH):
                pltpu.sync_copy(msg, tsh.at[pl.ds(s * BS + r * KCH, KCH)])
        plsc.subcore_barrier()

        def twork(b4):
            if not with_t:
                return
            saved = []
            for g in range(KCH // LL):
                lid = lanes + (g * LL)
                rv = rbuf[b4][pl.ds(g * LL, LL)]
                dv = plsc.load_gather(disv, [rv])
                e0 = plsc.load_gather(ebuf[b4], [lid, zi])
                e1 = plsc.load_gather(ebuf[b4], [lid, oi])
                plsc.store_scatter(msg, [lid, e0], dv)
                plsc.store_scatter(msg, [lid, e1 + 6], dv)
                saved.append((lid, e0, e1))
            pltpu.sync_copy(msg, tsh.at[cbuf[b4]], add=True)
            for (lid, e0, e1) in saved:
                plsc.store_scatter(msg, [lid, e0], z16)
                plsc.store_scatter(msg, [lid, e1 + 6], z16)

        def step(j, _):
            for b in range(4):
                base = wid * EW + (4 * j + b) * KCH
                pltpu.sync_copy(row_hbm.at[pl.ds(base, KCH)], rbuf[b])
                pltpu.sync_copy(col_hbm.at[pl.ds(base, KCH)], cbuf[b])
                if with_t:
                    pltpu.sync_copy(ea_hbm.at[pl.ds(base, KCH)], ebuf[b])
                twork(b)
                gather(b, 0).wait()
                pltpu.sync_copy(rows[0], acc.at[cbuf[b]], add=True)
            return 0
        lax.fori_loop(0, NCHUNK // 4, step, 0)
        base = wid * EW + (NCHUNK - 1) * KCH
        pltpu.sync_copy(row_hbm.at[pl.ds(base, KCH)], rbuf[0])
        pltpu.sync_copy(col_hbm.at[pl.ds(base, KCH)], cbuf[0])
        if with_t:
            pltpu.sync_copy(ea_hbm.at[pl.ds(base, KCH)], ebuf[0])
        twork(0)
        gather(0, 0).wait()
        pltpu.sync_copy(rows[0], acc.at[cbuf[0]], add=True)

        plsc.subcore_barrier()
        pltpu.sync_copy(acc.at[pl.ds(s * BS, BS)],
                        out_hbm.at[c, pl.ds(s * BS, BS)])
        if with_t:
            pltpu.sync_copy(tsh.at[pl.ds(s * BS, BS)],
                            tout_hbm.at[c, pl.ds(s * BS, BS)])

    return _spmm


@functools.partial(
    pl.kernel,
    out_type=jax.ShapeDtypeStruct((NP,), jnp.float32),
    mesh=_mesh,
    compiler_params=pltpu.CompilerParams(use_tc_tiling_on_sc=False, needs_layout_passes=False),
    scratch_types=[
        pltpu.VMEM((BS + LL,), jnp.int32),
        pltpu.VMEM((BS + LL,), jnp.float32),
        pltpu.VMEM((BS + LL,), jnp.float32),
        pltpu.VMEM((BS,), jnp.float32),
        pltpu.VMEM((GSEG + 128,), jnp.float32),
        pltpu.VMEM((GSEG + 128,), jnp.float32),
        pltpu.VMEM((GSEG,), jnp.float32),
        pltpu.VMEM((GSEG,), jnp.float32),
        pltpu.VMEM((NS, GSEG), jnp.float32),
        pltpu.VMEM_SHARED((NS, GSEG), jnp.float32),
        pltpu.VMEM_SHARED((NS, GSEG), jnp.float32),
    ],
)
def _softmax_kernel(nr_hbm, batch_hbm, out_hbm,
                    bbuf, vflat, exbuf, obuf, smax, ssum, gmax, gsum,
                    call, shmax, shsum):
    c = lax.axis_index("c")
    s = lax.axis_index("s")
    lanes = jnp.arange(LL, dtype=jnp.int32)
    z16 = jnp.zeros((LL,), jnp.float32)
    ninf = jnp.full((LL,), -3.0e38, jnp.float32)
    l0m = lanes == 0

    pltpu.sync_copy(nr_hbm.at[pl.ds(s * BS, BS)], vflat.at[pl.ds(0, BS)])
    pltpu.sync_copy(batch_hbm.at[pl.ds(s * BS, BS)], bbuf.at[pl.ds(0, BS)])

    for g in range((GSEG + LL) // LL):
        smax[pl.ds(g * LL, LL)] = ninf

    def mloop(i, _):
        b = bbuf[pl.ds(i, LL)][0]
        v = vflat[pl.ds(i, LL)][0]
        w = smax[pl.ds(b, LL)]
        smax[pl.ds(b, LL)] = jnp.where(
            l0m, jnp.maximum(w, lax.full((LL,), v, jnp.float32)), w)
        return 0
    lax.fori_loop(0, BS, mloop, 0)

    pltpu.sync_copy(smax.at[pl.ds(0, GSEG)], shmax.at[s])
    plsc.subcore_barrier()
    pltpu.sync_copy(shmax, call)
    for g in range(GSEG // LL):
        acc = call[0, pl.ds(g * LL, LL)]
        for t in range(1, NS):
            acc = jnp.maximum(acc, call[t, pl.ds(g * LL, LL)])
        gmax[pl.ds(g * LL, LL)] = acc

    for g in range(BS // LL):
        vv = vflat[pl.ds(g * LL, LL)]
        bv = bbuf[pl.ds(g * LL, LL)]
        mg = plsc.load_gather(gmax, [bv])
        exbuf[pl.ds(g * LL, LL)] = jnp.exp(vv - mg)

    for g in range((GSEG + LL) // LL):
        ssum[pl.ds(g * LL, LL)] = z16

    def sloop(i, _):
        b = bbuf[pl.ds(i, LL)][0]
        ev = exbuf[pl.ds(i, LL)][0]
        w = ssum[pl.ds(b, LL)]
        ssum[pl.ds(b, LL)] = jnp.where(
            l0m, w + lax.full((LL,), ev, jnp.float32), w)
        return 0
    lax.fori_loop(0, BS, sloop, 0)

    pltpu.sync_copy(ssum.at[pl.ds(0, GSEG)], shsum.at[s])
    plsc.subcore_barrier()
    pltpu.sync_copy(shsum, call)
    for g in range(GSEG // LL):
        acc = call[0, pl.ds(g * LL, LL)]
        for t in range(1, NS):
            acc = acc + call[t, pl.ds(g * LL, LL)]
        gsum[pl.ds(g * LL, LL)] = acc

    for g in range(BS // LL):
        bv = bbuf[pl.ds(g * LL, LL)]
        sv = plsc.load_gather(gsum, [bv])
        ev = exbuf[pl.ds(g * LL, LL)]
        obuf[pl.ds(g * LL, LL)] = ev / (sv + 1e-16)

    @pl.when(jnp.logical_and(s >= c * 8, s < c * 8 + 8))
    def _():
        pltpu.sync_copy(obuf, out_hbm.at[pl.ds(s * BS, BS)])


def _embed_call(x, cnt2, xe1, xe2, w1, b1):
    v1 = xe1.shape[0]
    v2 = xe2.shape[0]
    emb = xe1.shape[1]
    d1 = w1.shape[1]

    def body(x_ref, cnt_ref, xe1_ref, xe2_ref, w_ref, b_ref, hxp_ref, dis_ref):
        i = pl.program_id(0)
        xb = x_ref[...]
        cb = cnt_ref[:, pl.ds(i * ROWB, ROWB), :]
        deg = cb[0] + cb[1] + 1.0
        dis = lax.rsqrt(deg)
        oh0 = (xb[:, 0:1] == lax.broadcasted_iota(jnp.int32, (1, v1), 1)
               ).astype(jnp.float32)
        oh1 = (xb[:, 1:2] == lax.broadcasted_iota(jnp.int32, (1, v2), 1)
               ).astype(jnp.float32)
        h0 = (jnp.dot(oh0, xe1_ref[...], preferred_element_type=jnp.float32)
              + jnp.dot(oh1, xe2_ref[...], preferred_element_type=jnp.float32))
        hx = jnp.dot(h0, w_ref[...], preferred_element_type=jnp.float32) \
            + b_ref[...]
        hxp_ref[...] = dis * hx
        dis_ref[...] = dis

    return pl.pallas_call(
        body,
        grid=(N // ROWB,),
        in_specs=[
            pl.BlockSpec((ROWB, 2), lambda i: (i, 0)),
            pl.BlockSpec((NC, NP, 1), lambda i: (0, 0, 0)),
            pl.BlockSpec((v1, emb), lambda i: (0, 0)),
            pl.BlockSpec((v2, emb), lambda i: (0, 0)),
            pl.BlockSpec((emb, d1), lambda i: (0, 0)),
            pl.BlockSpec((1, d1), lambda i: (0, 0)),
        ],
        out_specs=[
            pl.BlockSpec((ROWB, d1), lambda i: (i, 0)),
            pl.BlockSpec((ROWB, 1), lambda i: (i, 0)),
        ],
        out_shape=[
            jax.ShapeDtypeStruct((N, d1), jnp.float32),
            jax.ShapeDtypeStruct((N, 1), jnp.float32),
        ],
    )(x, cnt2, xe1, xe2, w1, b1)


def _zstats_call(acc2, hxp, dis, t2, ecat):
    d = hxp.shape[1]

    def body(acc_ref, hxp_ref, dis_ref, t_ref, ec_ref, z_ref, st_ref):
        i = pl.program_id(0)
        dis_b = dis_ref[...]
        ec = ec_ref[...]
        tb = t_ref[0] + t_ref[1]
        ee = jnp.dot(tb, ec, preferred_element_type=jnp.float32)
        sl = (ec[4, :] + ec[6, :])[None, :]
        z = dis_b * (acc_ref[0] + acc_ref[1] + hxp_ref[...] + ee + dis_b * sl)
        z_ref[...] = z
        ps = jnp.concatenate(
            [jnp.sum(z, axis=0, keepdims=True),
             jnp.sum(z * z, axis=0, keepdims=True)], axis=0)
        st_ref[...] = jnp.where(i == 0, ps, st_ref[...] + ps)

    return pl.pallas_call(
        body,
        grid=(N // ROWB,),
        in_specs=[
            pl.BlockSpec((NC, ROWB, d), lambda i: (0, i, 0)),
            pl.BlockSpec((ROWB, d), lambda i: (i, 0)),
            pl.BlockSpec((ROWB, 1), lambda i: (i, 0)),
            pl.BlockSpec((NC, ROWB, 16), lambda i: (0, i, 0)),
            pl.BlockSpec((16, d), lambda i: (0, 0)),
        ],
        out_specs=[
            pl.BlockSpec((ROWB, d), lambda i: (i, 0)),
            pl.BlockSpec((2, d), lambda i: (0, 0)),
        ],
        out_shape=[
            jax.ShapeDtypeStruct((N, d), jnp.float32),
            jax.ShapeDtypeStruct((2, d), jnp.float32),
        ],
    )(acc2, hxp, dis, t2, ecat)


def _bn_mm_call(z, stats, gamma, beta, wn, bn, dis, relu, final):
    d = z.shape[1]
    dn = wn.shape[1]
    inv_n = 1.0 / float(N)

    def body(z_ref, st_ref, g_ref, be_ref, w_ref, b_ref, dis_ref, out_ref):
        st = st_ref[...]
        mean = st[0:1, :] * inv_n
        var = st[1:2, :] * inv_n - mean * mean
        scale = g_ref[...] * lax.rsqrt(var + 1e-5)
        shift = be_ref[...] - mean * scale
        h = z_ref[...] * scale + shift
        if relu:
            h = jnp.maximum(h, 0.0)
        o = jnp.dot(h, w_ref[...], preferred_element_type=jnp.float32) \
            + b_ref[...]
        out_ref[...] = o if final else dis_ref[...] * o

    return pl.pallas_call(
        body,
        grid=(N // ROWB,),
        in_specs=[
            pl.BlockSpec((ROWB, d), lambda i: (i, 0)),
            pl.BlockSpec((2, d), lambda i: (0, 0)),
            pl.BlockSpec((1, d), lambda i: (0, 0)),
            pl.BlockSpec((1, d), lambda i: (0, 0)),
            pl.BlockSpec((d, dn), lambda i: (0, 0)),
            pl.BlockSpec((1, dn), lambda i: (0, 0)),
            pl.BlockSpec((ROWB, 1), lambda i: (i, 0)),
        ],
        out_specs=pl.BlockSpec((ROWB, dn), lambda i: (i, 0)),
        out_shape=jax.ShapeDtypeStruct((N, dn), jnp.float32),
    )(z, stats, gamma, beta, wn, bn, dis)


_spmm_cache = {}


def _spmm_call(hxp, rowr, colr, ear=None, dis1=None):
    d = hxp.shape[1]
    with_t = ear is not None
    key = (d, with_t)
    if key not in _spmm_cache:
        _spmm_cache[key] = _make_spmm(d, with_t)
    if with_t:
        return _spmm_cache[key](hxp, rowr, colr, ear, dis1)
    return _spmm_cache[key](hxp, rowr, colr)[0]


def kernel(x, edge_index, edge_attr, batch, params):
    row = edge_index[0]
    col = edge_index[1]
    layers = params['layers']

    cnt2 = _deg_kernel(row).reshape(NC, NP, 1)

    l0 = layers[0]
    hxp, dis = _embed_call(x, cnt2, params['xe1'], params['xe2'],
                           l0['W'], l0['b'].reshape(1, -1))

    rowr = row
    colr = col
    ear = edge_attr

    nl = len(layers)
    nr = None
    t2 = None
    for li in range(nl):
        p = layers[li]
        d = p['W'].shape[1]
        if li == 0:
            acc2, t2 = _spmm_call(hxp, rowr, colr, ear, dis[:, 0])
        else:
            acc2 = _spmm_call(hxp, rowr, colr)
        ecat = jnp.concatenate(
            [p['e1'], p['e2'],
             jnp.zeros((16 - p['e1'].shape[0] - p['e2'].shape[0], d),
                       jnp.float32)], axis=0)
        z, stats = _zstats_call(acc2, hxp, dis, t2, ecat)
        if li < nl - 1:
            pn = layers[li + 1]
            hxp = _bn_mm_call(z, stats, p['gamma'].reshape(1, -1),
                              p['beta'].reshape(1, -1), pn['W'],
                              pn['b'].reshape(1, -1), dis,
                              relu=True, final=False)
        else:
            nr = _bn_mm_call(z, stats, p['gamma'].reshape(1, -1),
                             p['beta'].reshape(1, -1), params['Wf'],
                             params['bf'].reshape(1, -1), dis,
                             relu=False, final=True)

    nrp = jnp.pad(nr[:, 0], (0, NP - N), constant_values=-3.0e38)
    batchp = jnp.pad(batch, (0, NP - N))
    return _softmax_kernel(nrp, batchp)[:N, None]


# speedup vs baseline: 14.2234x; 1.5440x over previous
import functools
import jax
import jax.numpy as jnp
from jax import lax
from jax.experimental import pallas as pl
from jax.experimental.pallas import tpu as pltpu
from jax.experimental.pallas import tpu_sc as plsc

N = 10000
E = 320000
NC, NS, LL = 2, 16, 16
NW = NC * NS
EW = E // NW
KCH = 80
NCHUNK = EW // KCH
NP = 10240
BS = NP // NS
GSEG = 256
ROWB = 400

_mesh = plsc.VectorSubcoreMesh(core_axis_name="c", subcore_axis_name="s",
                               num_cores=NC, num_subcores=NS)


@functools.partial(
    pl.kernel,
    out_type=jax.ShapeDtypeStruct((NC * NP,), jnp.float32),
    mesh=_mesh,
    compiler_params=pltpu.CompilerParams(use_tc_tiling_on_sc=False, needs_layout_passes=False),
    scratch_types=[
        pltpu.VMEM((EW + LL,), jnp.int32),
        pltpu.VMEM((NP + LL,), jnp.float32),
        pltpu.VMEM((NS, BS), jnp.float32),
        pltpu.VMEM((BS,), jnp.float32),
        pltpu.VMEM_SHARED((NS, NP), jnp.float32),
    ],
)
def _deg_kernel(row_hbm, out_hbm, ridx, cnt, comb, res, shared):
    c = lax.axis_index("c")
    s = lax.axis_index("s")
    wid = s * NC + c
    z16 = jnp.zeros((LL,), jnp.float32)
    lanes = jnp.arange(LL, dtype=jnp.int32)
    oneh = jnp.where(lanes == 0, 1.0, 0.0).astype(jnp.float32)

    def zb(i, _):
        cnt[pl.ds(i * LL, LL)] = z16
        return 0
    lax.fori_loop(0, (NP + LL) // LL, zb, 0)

    pltpu.sync_copy(row_hbm.at[pl.ds(wid * EW, EW)], ridx.at[pl.ds(0, EW)])

    def body(j, _):
        rv = ridx[pl.ds(j * LL, LL)]
        for l in range(LL):
            r = rv[l]
            w = cnt[pl.ds(r, LL)]
            cnt[pl.ds(r, LL)] = w + oneh
        return 0
    lax.fori_loop(0, EW // LL, body, 0)

    pltpu.sync_copy(cnt.at[pl.ds(0, NP)], shared.at[s])
    plsc.subcore_barrier()
    pltpu.sync_copy(shared.at[:, pl.ds(s * BS, BS)], comb)

    for j in range(BS // LL):
        acc = comb[0, pl.ds(j * LL, LL)]
        for t in range(1, NS):
            acc = acc + comb[t, pl.ds(j * LL, LL)]
        res[pl.ds(j * LL, LL)] = acc
    pltpu.sync_copy(res, out_hbm.at[pl.ds(c * NP + s * BS, BS)])


def _make_spmm(d, with_t):
    zr = 32
    out_types = [jax.ShapeDtypeStruct((NC, NP, d), jnp.float32)]
    scratch = [
        [pltpu.VMEM((KCH,), jnp.int32) for _ in range(4)],
        [pltpu.VMEM((KCH,), jnp.int32) for _ in range(4)],
        [pltpu.SemaphoreType.DMA for _ in range(4)],
        [pltpu.VMEM((KCH, d), jnp.float32) for _ in range(2)],
        [pltpu.SemaphoreType.DMA for _ in range(2)],
        pltpu.VMEM((zr, d), jnp.float32),
        pltpu.VMEM_SHARED((NP, d), jnp.float32),
    ]
    if with_t:
        out_types.append(jax.ShapeDtypeStruct((NC, NP, 16), jnp.float32))
        scratch += [
            pltpu.VMEM((NP,), jnp.float32),
            [pltpu.VMEM((KCH, 2), jnp.int32) for _ in range(4)],
            pltpu.VMEM((KCH, 16), jnp.float32),
            pltpu.VMEM_SHARED((NP, 16), jnp.float32),
        ]

    @functools.partial(
        pl.kernel,
        out_type=tuple(out_types),
        mesh=_mesh,
        compiler_params=pltpu.CompilerParams(use_tc_tiling_on_sc=False,
                                             needs_layout_passes=False),
        scratch_types=scratch,
    )
    def _spmm(hxp_hbm, row_hbm, col_hbm, *rest):
        if with_t:
            (ea_hbm, dis_hbm, out_hbm, tout_hbm, rbuf, cbuf, semi, rows,
             semg, zbuf, acc, disv, ebuf, msg, tsh) = rest
        else:
            (out_hbm, rbuf, cbuf, semi, rows, semg, zbuf, acc) = rest
        c = lax.axis_index("c")
        s = lax.axis_index("s")
        wid = s * NC + c
        lanes = jnp.arange(LL, dtype=jnp.int32)
        zi = jnp.zeros((LL,), jnp.int32)
        oi = jnp.full((LL,), 1, jnp.int32)
        z16 = jnp.zeros((LL,), jnp.float32)

        def idx_fill(cch, p):
            base = jnp.minimum(wid * EW + cch * KCH, E - KCH)
            pltpu.async_copy(row_hbm.at[pl.ds(base, KCH)], rbuf[p], semi[p])
            pltpu.async_copy(col_hbm.at[pl.ds(base, KCH)], cbuf[p], semi[p])
            if with_t:
                pltpu.async_copy(ea_hbm.at[pl.ds(base, KCH)],
                                 ebuf[p], semi[p])

        def idx_wait(p):
            pltpu.make_async_copy(row_hbm.at[pl.ds(0, KCH)],
                                  rbuf[p], semi[p]).wait()
            pltpu.make_async_copy(col_hbm.at[pl.ds(0, KCH)],
                                  cbuf[p], semi[p]).wait()
            if with_t:
                pltpu.make_async_copy(ea_hbm.at[pl.ds(0, KCH)],
                                      ebuf[p], semi[p]).wait()

        def gather(p, b):
            return pltpu.async_copy(hxp_hbm.at[rbuf[p]], rows[b], semg[b])

        for p in range(4):
            idx_fill(p, p)

        if with_t:
            pltpu.sync_copy(dis_hbm, disv.at[pl.ds(0, N)])

        def zrow(i, _):
            for j in range(d // LL):
                zbuf[i, pl.ds(j * LL, LL)] = z16
            return 0
        lax.fori_loop(0, zr, zrow, 0)
        for r in range(BS // zr):
            pltpu.sync_copy(zbuf, acc.at[pl.ds(s * BS + r * zr, zr)])
        if with_t:
            def zmsg(i, _):
                msg[i, :] = z16
                return 0
            lax.fori_loop(0, KCH, zmsg, 0)
            for r in range(BS // KCH):
                pltpu.sync_copy(msg, tsh.at[pl.ds(s * BS + r * KCH, KCH)])
        plsc.subcore_barrier()

        def twork(b4):
            if not with_t:
                return
            saved = []
            for g in range(KCH // LL):
                lid = lanes + (g * LL)
                rv = rbuf[b4][pl.ds(g * LL, LL)]
                dv = plsc.load_gather(disv, [rv])
                e0 = plsc.load_gather(ebuf[b4], [lid, zi])
                e1 = plsc.load_gather(ebuf[b4], [lid, oi])
                plsc.store_scatter(msg, [lid, e0], dv)
                plsc.store_scatter(msg, [lid, e1 + 6], dv)
                saved.append((lid, e0, e1))
            pltpu.sync_copy(msg, tsh.at[cbuf[b4]], add=True)
            for (lid, e0, e1) in saved:
                plsc.store_scatter(msg, [lid, e0], z16)
                plsc.store_scatter(msg, [lid, e1 + 6], z16)

        if with_t:
            def step(j, _):
                c0 = 4 * j
                for b in range(4):
                    idx_wait(b)
                    dg = gather(b, 0)
                    twork(b)
                    dg.wait()
                    pltpu.sync_copy(rows[0], acc.at[cbuf[b]], add=True)
                    idx_fill(c0 + 4 + b, b)
                return 0
            lax.fori_loop(0, NCHUNK // 4, step, 0)
            idx_wait(0)
            dg = gather(0, 0)
            twork(0)
            dg.wait()
            pltpu.sync_copy(rows[0], acc.at[cbuf[0]], add=True)
            idx_wait(1)
            idx_wait(2)
            idx_wait(3)
        else:
            def step(j, _):
                c0 = 4 * j
                idx_wait(0)
                d0 = gather(0, 0)
                idx_wait(1)
                d1 = gather(1, 1)
                d0.wait()
                pltpu.sync_copy(rows[0], acc.at[cbuf[0]], add=True)
                idx_wait(2)
                d2 = gather(2, 0)
                d1.wait()
                pltpu.sync_copy(rows[1], acc.at[cbuf[1]], add=True)
                idx_fill(c0 + 4, 0)
                idx_fill(c0 + 5, 1)
                idx_wait(3)
                d3 = gather(3, 1)
                d2.wait()
                pltpu.sync_copy(rows[0], acc.at[cbuf[2]], add=True)
                d3.wait()
                pltpu.sync_copy(rows[1], acc.at[cbuf[3]], add=True)
                idx_fill(c0 + 6, 2)
                idx_fill(c0 + 7, 3)
                return 0
            lax.fori_loop(0, NCHUNK // 4, step, 0)
            idx_wait(0)
            d0 = gather(0, 0)
            d0.wait()
            pltpu.sync_copy(rows[0], acc.at[cbuf[0]], add=True)
            idx_wait(1)
            idx_wait(2)
            idx_wait(3)

        plsc.subcore_barrier()
        pltpu.sync_copy(acc.at[pl.ds(s * BS, BS)],
                        out_hbm.at[c, pl.ds(s * BS, BS)])
        if with_t:
            pltpu.sync_copy(tsh.at[pl.ds(s * BS, BS)],
                            tout_hbm.at[c, pl.ds(s * BS, BS)])

    return _spmm


@functools.partial(
    pl.kernel,
    out_type=jax.ShapeDtypeStruct((NP,), jnp.float32),
    mesh=_mesh,
    compiler_params=pltpu.CompilerParams(use_tc_tiling_on_sc=False, needs_layout_passes=False),
    scratch_types=[
        pltpu.VMEM((BS + LL,), jnp.int32),
        pltpu.VMEM((BS + LL,), jnp.float32),
        pltpu.VMEM((BS + LL,), jnp.float32),
        pltpu.VMEM((BS,), jnp.float32),
        pltpu.VMEM((GSEG + 128,), jnp.float32),
        pltpu.VMEM((GSEG + 128,), jnp.float32),
        pltpu.VMEM((GSEG,), jnp.float32),
        pltpu.VMEM((GSEG,), jnp.float32),
        pltpu.VMEM((NS, GSEG), jnp.float32),
        pltpu.VMEM_SHARED((NS, GSEG), jnp.float32),
        pltpu.VMEM_SHARED((NS, GSEG), jnp.float32),
    ],
)
def _softmax_kernel(nr_hbm, batch_hbm, out_hbm,
                    bbuf, vflat, exbuf, obuf, smax, ssum, gmax, gsum,
                    call, shmax, shsum):
    c = lax.axis_index("c")
    s = lax.axis_index("s")
    lanes = jnp.arange(LL, dtype=jnp.int32)
    z16 = jnp.zeros((LL,), jnp.float32)
    ninf = jnp.full((LL,), -3.0e38, jnp.float32)
    l0m = lanes == 0

    pltpu.sync_copy(nr_hbm.at[pl.ds(s * BS, BS)], vflat.at[pl.ds(0, BS)])
    pltpu.sync_copy(batch_hbm.at[pl.ds(s * BS, BS)], bbuf.at[pl.ds(0, BS)])

    for g in range((GSEG + LL) // LL):
        smax[pl.ds(g * LL, LL)] = ninf

    def mloop(i, _):
        b = bbuf[pl.ds(i, LL)][0]
        v = vflat[pl.ds(i, LL)][0]
        w = smax[pl.ds(b, LL)]
        smax[pl.ds(b, LL)] = jnp.where(
            l0m, jnp.maximum(w, lax.full((LL,), v, jnp.float32)), w)
        return 0
    lax.fori_loop(0, BS, mloop, 0)

    pltpu.sync_copy(smax.at[pl.ds(0, GSEG)], shmax.at[s])
    plsc.subcore_barrier()
    pltpu.sync_copy(shmax, call)
    for g in range(GSEG // LL):
        acc = call[0, pl.ds(g * LL, LL)]
        for t in range(1, NS):
            acc = jnp.maximum(acc, call[t, pl.ds(g * LL, LL)])
        gmax[pl.ds(g * LL, LL)] = acc

    for g in range(BS // LL):
        vv = vflat[pl.ds(g * LL, LL)]
        bv = bbuf[pl.ds(g * LL, LL)]
        mg = plsc.load_gather(gmax, [bv])
        exbuf[pl.ds(g * LL, LL)] = jnp.exp(vv - mg)

    for g in range((GSEG + LL) // LL):
        ssum[pl.ds(g * LL, LL)] = z16

    def sloop(i, _):
        b = bbuf[pl.ds(i, LL)][0]
        ev = exbuf[pl.ds(i, LL)][0]
        w = ssum[pl.ds(b, LL)]
        ssum[pl.ds(b, LL)] = jnp.where(
            l0m, w + lax.full((LL,), ev, jnp.float32), w)
        return 0
    lax.fori_loop(0, BS, sloop, 0)

    pltpu.sync_copy(ssum.at[pl.ds(0, GSEG)], shsum.at[s])
    plsc.subcore_barrier()
    pltpu.sync_copy(shsum, call)
    for g in range(GSEG // LL):
        acc = call[0, pl.ds(g * LL, LL)]
        for t in range(1, NS):
            acc = acc + call[t, pl.ds(g * LL, LL)]
        gsum[pl.ds(g * LL, LL)] = acc

    for g in range(BS // LL):
        bv = bbuf[pl.ds(g * LL, LL)]
        sv = plsc.load_gather(gsum, [bv])
        ev = exbuf[pl.ds(g * LL, LL)]
        obuf[pl.ds(g * LL, LL)] = ev / (sv + 1e-16)

    @pl.when(jnp.logical_and(s >= c * 8, s < c * 8 + 8))
    def _():
        pltpu.sync_copy(obuf, out_hbm.at[pl.ds(s * BS, BS)])


def _embed_call(x, cnt2, xe1, xe2, w1, b1):
    v1 = xe1.shape[0]
    v2 = xe2.shape[0]
    emb = xe1.shape[1]
    d1 = w1.shape[1]

    def body(x_ref, cnt_ref, xe1_ref, xe2_ref, w_ref, b_ref, hxp_ref, dis_ref):
        i = pl.program_id(0)
        xb = x_ref[...]
        cb = cnt_ref[:, pl.ds(i * ROWB, ROWB), :]
        deg = cb[0] + cb[1] + 1.0
        dis = lax.rsqrt(deg)
        oh0 = (xb[:, 0:1] == lax.broadcasted_iota(jnp.int32, (1, v1), 1)
               ).astype(jnp.float32)
        oh1 = (xb[:, 1:2] == lax.broadcasted_iota(jnp.int32, (1, v2), 1)
               ).astype(jnp.float32)
        h0 = (jnp.dot(oh0, xe1_ref[...], preferred_element_type=jnp.float32)
              + jnp.dot(oh1, xe2_ref[...], preferred_element_type=jnp.float32))
        hx = jnp.dot(h0, w_ref[...], preferred_element_type=jnp.float32) \
            + b_ref[...]
        hxp_ref[...] = dis * hx
        dis_ref[...] = dis

    return pl.pallas_call(
        body,
        grid=(N // ROWB,),
        in_specs=[
            pl.BlockSpec((ROWB, 2), lambda i: (i, 0)),
            pl.BlockSpec((NC, NP, 1), lambda i: (0, 0, 0)),
            pl.BlockSpec((v1, emb), lambda i: (0, 0)),
            pl.BlockSpec((v2, emb), lambda i: (0, 0)),
            pl.BlockSpec((emb, d1), lambda i: (0, 0)),
            pl.BlockSpec((1, d1), lambda i: (0, 0)),
        ],
        out_specs=[
            pl.BlockSpec((ROWB, d1), lambda i: (i, 0)),
            pl.BlockSpec((ROWB, 1), lambda i: (i, 0)),
        ],
        out_shape=[
            jax.ShapeDtypeStruct((N, d1), jnp.float32),
            jax.ShapeDtypeStruct((N, 1), jnp.float32),
        ],
    )(x, cnt2, xe1, xe2, w1, b1)


def _zstats_call(acc2, hxp, dis, t2, ecat):
    d = hxp.shape[1]

    def body(acc_ref, hxp_ref, dis_ref, t_ref, ec_ref, z_ref, st_ref):
        i = pl.program_id(0)
        dis_b = dis_ref[...]
        ec = ec_ref[...]
        tb = t_ref[0] + t_ref[1]
        ee = jnp.dot(tb, ec, preferred_element_type=jnp.float32)
        sl = (ec[4, :] + ec[6, :])[None, :]
        z = dis_b * (acc_ref[0] + acc_ref[1] + hxp_ref[...] + ee + dis_b * sl)
        z_ref[...] = z
        ps = jnp.concatenate(
            [jnp.sum(z, axis=0, keepdims=True),
             jnp.sum(z * z, axis=0, keepdims=True)], axis=0)
        st_ref[...] = jnp.where(i == 0, ps, st_ref[...] + ps)

    return pl.pallas_call(
        body,
        grid=(N // ROWB,),
        in_specs=[
            pl.BlockSpec((NC, ROWB, d), lambda i: (0, i, 0)),
            pl.BlockSpec((ROWB, d), lambda i: (i, 0)),
            pl.BlockSpec((ROWB, 1), lambda i: (i, 0)),
            pl.BlockSpec((NC, ROWB, 16), lambda i: (0, i, 0)),
            pl.BlockSpec((16, d), lambda i: (0, 0)),
        ],
        out_specs=[
            pl.BlockSpec((ROWB, d), lambda i: (i, 0)),
            pl.BlockSpec((2, d), lambda i: (0, 0)),
        ],
        out_shape=[
            jax.ShapeDtypeStruct((N, d), jnp.float32),
            jax.ShapeDtypeStruct((2, d), jnp.float32),
        ],
    )(acc2, hxp, dis, t2, ecat)


def _bn_mm_call(z, stats, gamma, beta, wn, bn, dis, relu, final):
    d = z.shape[1]
    dn = wn.shape[1]
    inv_n = 1.0 / float(N)

    def body(z_ref, st_ref, g_ref, be_ref, w_ref, b_ref, dis_ref, out_ref):
        st = st_ref[...]
        mean = st[0:1, :] * inv_n
        var = st[1:2, :] * inv_n - mean * mean
        scale = g_ref[...] * lax.rsqrt(var + 1e-5)
        shift = be_ref[...] - mean * scale
        h = z_ref[...] * scale + shift
        if relu:
            h = jnp.maximum(h, 0.0)
        o = jnp.dot(h, w_ref[...], preferred_element_type=jnp.float32) \
            + b_ref[...]
        out_ref[...] = o if final else dis_ref[...] * o

    return pl.pallas_call(
        body,
        grid=(N // ROWB,),
        in_specs=[
            pl.BlockSpec((ROWB, d), lambda i: (i, 0)),
            pl.BlockSpec((2, d), lambda i: (0, 0)),
            pl.BlockSpec((1, d), lambda i: (0, 0)),
            pl.BlockSpec((1, d), lambda i: (0, 0)),
            pl.BlockSpec((d, dn), lambda i: (0, 0)),
            pl.BlockSpec((1, dn), lambda i: (0, 0)),
            pl.BlockSpec((ROWB, 1), lambda i: (i, 0)),
        ],
        out_specs=pl.BlockSpec((ROWB, dn), lambda i: (i, 0)),
        out_shape=jax.ShapeDtypeStruct((N, dn), jnp.float32),
    )(z, stats, gamma, beta, wn, bn, dis)


_spmm_cache = {}


def _spmm_call(hxp, rowr, colr, ear=None, dis1=None):
    d = hxp.shape[1]
    with_t = ear is not None
    key = (d, with_t)
    if key not in _spmm_cache:
        _spmm_cache[key] = _make_spmm(d, with_t)
    if with_t:
        return _spmm_cache[key](hxp, rowr, colr, ear, dis1)
    return _spmm_cache[key](hxp, rowr, colr)[0]


def kernel(x, edge_index, edge_attr, batch, params):
    row = edge_index[0]
    col = edge_index[1]
    layers = params['layers']

    cnt2 = _deg_kernel(row).reshape(NC, NP, 1)

    l0 = layers[0]
    hxp, dis = _embed_call(x, cnt2, params['xe1'], params['xe2'],
                           l0['W'], l0['b'].reshape(1, -1))

    rowr = row
    colr = col
    ear = edge_attr

    nl = len(layers)
    nr = None
    t2 = None
    for li in range(nl):
        p = layers[li]
        d = p['W'].shape[1]
        if li == 0:
            acc2, t2 = _spmm_call(hxp, rowr, colr, ear, dis[:, 0])
        else:
            acc2 = _spmm_call(hxp, rowr, colr)
        ecat = jnp.concatenate(
            [p['e1'], p['e2'],
             jnp.zeros((16 - p['e1'].shape[0] - p['e2'].shape[0], d),
                       jnp.float32)], axis=0)
        z, stats = _zstats_call(acc2, hxp, dis, t2, ecat)
        if li < nl - 1:
            pn = layers[li + 1]
            hxp = _bn_mm_call(z, stats, p['gamma'].reshape(1, -1),
                              p['beta'].reshape(1, -1), pn['W'],
                              pn['b'].reshape(1, -1), dis,
                              relu=True, final=False)
        else:
            nr = _bn_mm_call(z, stats, p['gamma'].reshape(1, -1),
                             p['beta'].reshape(1, -1), params['Wf'],
                             params['bf'].reshape(1, -1), dis,
                             relu=False, final=True)

    nrp = jnp.pad(nr[:, 0], (0, NP - N), constant_values=-3.0e38)
    batchp = jnp.pad(batch, (0, NP - N))
    return _softmax_kernel(nrp, batchp)[:N, None]
